# Initial kernel scaffold; baseline (speedup 1.0000x reference)
import jax
import jax.numpy as jnp
from jax.experimental import pallas as pl


def kernel(x, Wg, W1, W3, W2):
    raise NotImplementedError("write your pallas kernel here")


# speedup vs baseline: 1.1031x; 1.1031x over previous
import functools

import jax
import jax.numpy as jnp
from jax.experimental import pallas as pl
from jax.experimental.pallas import tpu as pltpu

N, D = 2048, 768
E, K, H = 8, 2, 2048
TM = 256


def _router_body(x_ref, wg_ref, coef_ref):
    logits = jax.lax.dot_general(
        x_ref[...], wg_ref[...], (((1,), (1,)), ((), ())),
        preferred_element_type=jnp.float32,
    )
    e_iota = jax.lax.broadcasted_iota(jnp.int32, logits.shape, 1)
    big = jnp.int32(E + 1)
    top1 = jnp.max(logits, axis=-1, keepdims=True)
    a1 = jnp.min(jnp.where(logits == top1, e_iota, big), axis=-1, keepdims=True)
    m1 = e_iota == a1
    logits2 = jnp.where(m1, -jnp.inf, logits)
    top2 = jnp.max(logits2, axis=-1, keepdims=True)
    a2 = jnp.min(jnp.where(logits2 == top2, e_iota, big), axis=-1, keepdims=True)
    m2 = e_iota == a2
    z = jnp.exp(top2 - top1)
    w1 = 1.0 / (1.0 + z)
    w2 = z / (1.0 + z)
    routed = m1 | m2
    counts = jnp.sum(routed.astype(jnp.int32), axis=0, keepdims=True)
    t_iota = jax.lax.broadcasted_iota(jnp.int32, logits.shape, 0)
    bug = t_iota < counts
    weight = jnp.where(m1, w1, 0.0) + jnp.where(m2, w2, 0.0)
    coef_ref[...] = jnp.where(routed & bug, weight, jnp.float32(0.0))


def _ffn_body(x_ref, c_ref, w1_ref, w3_ref, w2_ref, o_ref):
    e = pl.program_id(1)
    xt = x_ref[...]
    h1 = jax.lax.dot_general(xt, w1_ref[0], (((1,), (1,)), ((), ())),
                             preferred_element_type=jnp.float32)
    h3 = jax.lax.dot_general(xt, w3_ref[0], (((1,), (1,)), ((), ())),
                             preferred_element_type=jnp.float32)
    h = (h1 * jax.nn.sigmoid(h1) * h3).astype(jnp.bfloat16)
    eo = jax.lax.dot_general(h, w2_ref[0], (((1,), (1,)), ((), ())),
                             preferred_element_type=jnp.float32)
    lane = jax.lax.broadcasted_iota(jnp.int32, (TM, E), 1)
    coef = jnp.sum(jnp.where(lane == e, c_ref[...], 0.0), axis=1,
                   keepdims=True)
    acc = coef * eo

    @pl.when(e == 0)
    def _():
        o_ref[...] = acc

    @pl.when(e != 0)
    def _():
        o_ref[...] = o_ref[...] + acc


@jax.jit
def kernel(x, Wg, W1, W3, W2):
    b, s, d = x.shape
    xf = x.reshape(N, D)

    coef = pl.pallas_call(
        _router_body,
        out_shape=jax.ShapeDtypeStruct((N, E), jnp.float32),
    )(xf, Wg)

    xb = xf.astype(jnp.bfloat16)
    w1b = W1.astype(jnp.bfloat16)
    w3b = W3.astype(jnp.bfloat16)
    w2b = W2.astype(jnp.bfloat16)

    out = pl.pallas_call(
        _ffn_body,
        grid=(N // TM, E),
        in_specs=[
            pl.BlockSpec((TM, D), lambda m, e: (m, 0)),
            pl.BlockSpec((TM, E), lambda m, e: (m, 0)),
            pl.BlockSpec((1, H, D), lambda m, e: (e, 0, 0)),
            pl.BlockSpec((1, H, D), lambda m, e: (e, 0, 0)),
            pl.BlockSpec((1, D, H), lambda m, e: (e, 0, 0)),
        ],
        out_specs=pl.BlockSpec((TM, D), lambda m, e: (m, 0)),
        out_shape=jax.ShapeDtypeStruct((N, D), jnp.float32),
        compiler_params=pltpu.CompilerParams(
            dimension_semantics=("parallel", "arbitrary"),
        ),
    )(xb, coef, w1b, w3b, w2b)

    return out.reshape(b, s, d)
